# baseline (speedup 1.0000x reference)
import functools

import jax
import jax.numpy as jnp
from jax import lax
from jax.experimental import pallas as pl
from jax.experimental.pallas import tpu as pltpu
from jax.experimental.pallas import tpu_sc as plsc

B, T, D = 32, 64, 512
EMB = 1024
CODE_DIM = 64
NUM_CODES = 1024
BETA = 0.25

N = B * T
TILE = 1024
GRID = N // TILE


def _vq_kernel(s_ref, ns_ref, ws_ref, bs_ref, wp_ref, bp_ref, cb_ref,
               enc_ref, idx_ref, cbt_ref, loss_ref, perp_ref,
               err_acc, cnt_acc, cbsq_s):
    i = pl.program_id(0)

    @pl.when(i == 0)
    def _init():
        err_acc[0, 0] = 0.0
        cnt_acc[...] = jnp.zeros_like(cnt_acc)
        cbt_ref[...] = cb_ref[...].T
        cbsq_s[...] = jnp.sum(cb_ref[...] * cb_ref[...], axis=0, keepdims=True)

    s = jnp.dot(s_ref[...], ws_ref[...], preferred_element_type=jnp.float32)
    ns = jnp.dot(ns_ref[...], ws_ref[...], preferred_element_type=jnp.float32)
    s = jax.nn.gelu(s + bs_ref[...])
    ns = jax.nn.gelu(ns + bs_ref[...])
    z = (jnp.dot(s, wp_ref[:EMB, :], preferred_element_type=jnp.float32)
         + jnp.dot(ns, wp_ref[EMB:, :], preferred_element_type=jnp.float32)
         + bp_ref[...])

    cross = jnp.dot(z, cb_ref[...], preferred_element_type=jnp.float32)
    z_sq = jnp.sum(z * z, axis=1, keepdims=True)
    dist = z_sq - 2.0 * cross + cbsq_s[...]

    idx = jnp.argmin(dist, axis=1).astype(jnp.int32)
    ids = jax.lax.broadcasted_iota(jnp.int32, (TILE, NUM_CODES), 1)
    enc = (ids == idx[:, None]).astype(jnp.float32)
    enc_ref[...] = enc
    idx_ref[...] = idx.reshape(1, 1, TILE)

    err_acc[0, 0] += jnp.sum(jnp.min(dist, axis=1))
    cnt_acc[...] += jnp.sum(enc, axis=0, keepdims=True)

    @pl.when(i == GRID - 1)
    def _fini():
        loss_ref[0, 0] = BETA * err_acc[0, 0] / (N * CODE_DIM)
        p = cnt_acc[...] / N
        perp_ref[0, 0] = jnp.exp(-jnp.sum(p * jnp.log(p + 1e-10)))


_SC_INFO = plsc.get_sparse_core_info()
_NW = _SC_INFO.num_cores * _SC_INFO.num_subcores
_TOK_W = N // _NW


def _sc_gather(cbt, idx_flat):
    mesh = plsc.VectorSubcoreMesh(core_axis_name="c", subcore_axis_name="s")

    @functools.partial(
        pl.kernel, mesh=mesh,
        out_type=jax.ShapeDtypeStruct((N, CODE_DIM), jnp.float32),
        compiler_params=pltpu.CompilerParams(use_tc_tiling_on_sc=False),
        scratch_types=[
            pltpu.VMEM((_TOK_W,), jnp.int32),
            pltpu.VMEM((_TOK_W, CODE_DIM), jnp.float32),
            pltpu.SemaphoreType.DMA,
        ],
    )
    def k(cbt_hbm, idx_hbm, out_hbm, idx_v, rows_v, sem):
        wid = lax.axis_index("s") * _SC_INFO.num_cores + lax.axis_index("c")
        base = wid * _TOK_W
        pltpu.sync_copy(idx_hbm.at[pl.ds(base, _TOK_W)], idx_v)
        pltpu.async_copy(cbt_hbm.at[idx_v], rows_v, sem).wait()
        pltpu.sync_copy(rows_v, out_hbm.at[pl.ds(base, _TOK_W)])

    return k(cbt, idx_flat)


@jax.jit
def kernel(states, next_states, W_s, b_s, W_p, b_p, codebook):
    s2 = states.reshape(N, D)
    ns2 = next_states.reshape(N, D)

    enc, idx, cbt, loss, perp = pl.pallas_call(
        _vq_kernel,
        grid=(GRID,),
        in_specs=[
            pl.BlockSpec((TILE, D), lambda i: (i, 0)),
            pl.BlockSpec((TILE, D), lambda i: (i, 0)),
            pl.BlockSpec((D, EMB), lambda i: (0, 0)),
            pl.BlockSpec((EMB,), lambda i: (0,)),
            pl.BlockSpec((2 * EMB, CODE_DIM), lambda i: (0, 0)),
            pl.BlockSpec((CODE_DIM,), lambda i: (0,)),
            pl.BlockSpec((CODE_DIM, NUM_CODES), lambda i: (0, 0)),
        ],
        out_specs=[
            pl.BlockSpec((TILE, NUM_CODES), lambda i: (i, 0)),
            pl.BlockSpec((1, 1, TILE), lambda i: (i, 0, 0)),
            pl.BlockSpec((NUM_CODES, CODE_DIM), lambda i: (0, 0)),
            pl.BlockSpec(memory_space=pltpu.SMEM),
            pl.BlockSpec(memory_space=pltpu.SMEM),
        ],
        out_shape=[
            jax.ShapeDtypeStruct((N, NUM_CODES), jnp.float32),
            jax.ShapeDtypeStruct((GRID, 1, TILE), jnp.int32),
            jax.ShapeDtypeStruct((NUM_CODES, CODE_DIM), jnp.float32),
            jax.ShapeDtypeStruct((1, 1), jnp.float32),
            jax.ShapeDtypeStruct((1, 1), jnp.float32),
        ],
        scratch_shapes=[
            pltpu.SMEM((1, 1), jnp.float32),
            pltpu.VMEM((1, NUM_CODES), jnp.float32),
            pltpu.VMEM((1, NUM_CODES), jnp.float32),
        ],
        compiler_params=pltpu.CompilerParams(
            dimension_semantics=("arbitrary",),
        ),
    )(s2, ns2, W_s, b_s, W_p, b_p, codebook)

    quant = _sc_gather(cbt, idx.reshape(N))
    quantize_st = quant.reshape(B, T, CODE_DIM)
    encoding_indices = idx.reshape(B, T)
    return quantize_st, loss.reshape(()), perp.reshape(()), enc, encoding_indices


# speedup vs baseline: 1.8663x; 1.8663x over previous
import jax
import jax.numpy as jnp
from jax.experimental import pallas as pl
from jax.experimental.pallas import tpu as pltpu

B, T, D = 32, 64, 512
EMB = 1024
CODE_DIM = 64
NUM_CODES = 1024
BETA = 0.25

N = B * T
TILE = 1024
GRID = N // TILE


def _vq_kernel(s_ref, ns_ref, ws_ref, bs_ref, wp_ref, bp_ref, cb_ref,
               quant_ref, enc_ref, idx_ref, loss_ref, perp_ref,
               err_acc, cnt_acc, cbt_s, cbsq_s):
    i = pl.program_id(0)

    @pl.when(i == 0)
    def _init():
        err_acc[0, 0] = 0.0
        cnt_acc[...] = jnp.zeros_like(cnt_acc)
        cbt_s[...] = cb_ref[...].T
        cbsq_s[...] = jnp.sum(cb_ref[...] * cb_ref[...], axis=0, keepdims=True)

    s = jnp.dot(s_ref[...], ws_ref[...], preferred_element_type=jnp.float32)
    ns = jnp.dot(ns_ref[...], ws_ref[...], preferred_element_type=jnp.float32)
    s = jax.nn.gelu(s + bs_ref[...])
    ns = jax.nn.gelu(ns + bs_ref[...])
    z = (jnp.dot(s, wp_ref[:EMB, :], preferred_element_type=jnp.float32)
         + jnp.dot(ns, wp_ref[EMB:, :], preferred_element_type=jnp.float32)
         + bp_ref[...])

    cross = jnp.dot(z, cb_ref[...], preferred_element_type=jnp.float32)
    z_sq = jnp.sum(z * z, axis=1, keepdims=True)
    dist = z_sq - 2.0 * cross + cbsq_s[...]

    idx = jnp.argmin(dist, axis=1).astype(jnp.int32)
    ids = jax.lax.broadcasted_iota(jnp.int32, (TILE, NUM_CODES), 1)
    enc = (ids == idx[:, None]).astype(jnp.float32)
    enc_ref[...] = enc
    idx_ref[...] = idx.reshape(1, 1, TILE)

    q = jnp.dot(enc, cbt_s[...], preferred_element_type=jnp.float32)
    quant_ref[...] = q

    diff = q - z
    err_acc[0, 0] += jnp.sum(diff * diff)
    cnt_acc[...] += jnp.sum(enc, axis=0, keepdims=True)

    @pl.when(i == GRID - 1)
    def _fini():
        loss_ref[0, 0] = BETA * err_acc[0, 0] / (N * CODE_DIM)
        p = cnt_acc[...] / N
        perp_ref[0, 0] = jnp.exp(-jnp.sum(p * jnp.log(p + 1e-10)))


@jax.jit
def kernel(states, next_states, W_s, b_s, W_p, b_p, codebook):
    s2 = states.reshape(N, D)
    ns2 = next_states.reshape(N, D)

    quant, enc, idx, loss, perp = pl.pallas_call(
        _vq_kernel,
        grid=(GRID,),
        in_specs=[
            pl.BlockSpec((TILE, D), lambda i: (i, 0)),
            pl.BlockSpec((TILE, D), lambda i: (i, 0)),
            pl.BlockSpec((D, EMB), lambda i: (0, 0)),
            pl.BlockSpec((EMB,), lambda i: (0,)),
            pl.BlockSpec((2 * EMB, CODE_DIM), lambda i: (0, 0)),
            pl.BlockSpec((CODE_DIM,), lambda i: (0,)),
            pl.BlockSpec((CODE_DIM, NUM_CODES), lambda i: (0, 0)),
        ],
        out_specs=[
            pl.BlockSpec((TILE, CODE_DIM), lambda i: (i, 0)),
            pl.BlockSpec((TILE, NUM_CODES), lambda i: (i, 0)),
            pl.BlockSpec((1, 1, TILE), lambda i: (i, 0, 0)),
            pl.BlockSpec(memory_space=pltpu.SMEM),
            pl.BlockSpec(memory_space=pltpu.SMEM),
        ],
        out_shape=[
            jax.ShapeDtypeStruct((N, CODE_DIM), jnp.float32),
            jax.ShapeDtypeStruct((N, NUM_CODES), jnp.float32),
            jax.ShapeDtypeStruct((GRID, 1, TILE), jnp.int32),
            jax.ShapeDtypeStruct((1, 1), jnp.float32),
            jax.ShapeDtypeStruct((1, 1), jnp.float32),
        ],
        scratch_shapes=[
            pltpu.SMEM((1, 1), jnp.float32),
            pltpu.VMEM((1, NUM_CODES), jnp.float32),
            pltpu.VMEM((NUM_CODES, CODE_DIM), jnp.float32),
            pltpu.VMEM((1, NUM_CODES), jnp.float32),
        ],
        compiler_params=pltpu.CompilerParams(
            dimension_semantics=("arbitrary",),
        ),
    )(s2, ns2, W_s, b_s, W_p, b_p, codebook)

    quantize_st = quant.reshape(B, T, CODE_DIM)
    encoding_indices = idx.reshape(B, T)
    return quantize_st, loss.reshape(()), perp.reshape(()), enc, encoding_indices
